# baseline (speedup 1.0000x reference)
import functools

import jax
import jax.numpy as jnp
from jax import lax
from jax.experimental import pallas as pl
from jax.experimental.pallas import tpu as pltpu
from jax.experimental.pallas import tpu_sc as plsc

NC = 2
NS = 16
NW = NC * NS

NG = 200000
SL = 6256
NGP = NW * SL

NI = 204800
PI = 6400
PIR = 50
NCL = 1024
PC = 32

NTOT = NI + NCL
D = 128

OUT_ROWS = NTOT + 256
TRASH = NTOT + 128

_mesh = plsc.VectorSubcoreMesh(
    core_axis_name="c", subcore_axis_name="s", num_cores=NC, num_subcores=NS
)

_i32 = jnp.int32


def _wid():
    return lax.axis_index("s") * NC + lax.axis_index("c")


def _al(x):
    return pl.multiple_of(x, 8)


@functools.partial(
    pl.kernel,
    out_type=[
        jax.ShapeDtypeStruct((NI,), _i32),
        jax.ShapeDtypeStruct((NC * NGP,), _i32),
    ],
    mesh=_mesh,
    compiler_params=pltpu.CompilerParams(needs_layout_passes=False),
    scratch_types=[
        pltpu.VMEM((PI,), _i32),
        pltpu.VMEM((PI,), _i32),
        pltpu.VMEM((PC,), _i32),
        pltpu.VMEM((PC,), _i32),
        pltpu.VMEM((PI,), _i32),
        pltpu.VMEM((SL,), _i32),
        pltpu.VMEM_SHARED((NGP,), _i32),
        pltpu.SemaphoreType.DMA,
        pltpu.SemaphoreType.DMA,
    ],
)
def _k1(item_ids_h, client_ids_h, itab_h, ctab_h, gitems_h, counts_h,
        iidx_v, gitem_v, cidx_v, gcli_v, ones_v, zb_v, cnt_sh, sem, sem2):
    cid = lax.axis_index("c")
    sid = lax.axis_index("s")
    wid = sid * NC + cid

    def zfill(k, _):
        zb_v[pl.ds(k * 16, 16)] = jnp.zeros((16,), _i32)
        return 0
    lax.fori_loop(0, SL // 16, zfill, 0)
    pltpu.sync_copy(zb_v, cnt_sh.at[pl.ds(_al(sid * 2 * SL), SL)])
    pltpu.sync_copy(zb_v, cnt_sh.at[pl.ds(_al(sid * 2 * SL + SL), SL)])

    def ofill(k, _):
        ones_v[pl.ds(k * 16, 16)] = jnp.ones((16,), _i32)
        return 0
    lax.fori_loop(0, PI // 16, ofill, 0)

    pltpu.sync_copy(item_ids_h.at[pl.ds(_al(wid * PI), PI)], iidx_v)
    gdesc = pltpu.async_copy(itab_h.at[iidx_v], gitem_v, sem)

    pltpu.sync_copy(client_ids_h.at[pl.ds(_al(wid * PC), PC)], cidx_v)
    pltpu.async_copy(ctab_h.at[cidx_v], gcli_v, sem2).wait()

    gdesc.wait()
    pltpu.async_copy(gitem_v, gitems_h.at[pl.ds(_al(wid * PI), PI)], sem2)

    plsc.subcore_barrier()

    pltpu.async_copy(ones_v, cnt_sh.at[gitem_v], sem, add=True)
    pltpu.async_copy(ones_v.at[pl.ds(0, PC)], cnt_sh.at[gcli_v], sem,
                     add=True)
    pltpu.make_async_copy(itab_h.at[pl.ds(0, PC)], cidx_v, sem).wait()
    pltpu.make_async_copy(itab_h.at[pl.ds(0, PI)], ones_v, sem).wait()
    pltpu.make_async_copy(gitem_v, gitems_h.at[pl.ds(0, PI)], sem2).wait()

    plsc.subcore_barrier()

    pltpu.sync_copy(cnt_sh.at[pl.ds(_al(sid * 2 * SL), SL)], zb_v)
    pltpu.sync_copy(zb_v, counts_h.at[pl.ds(_al(cid * NGP + sid * 2 * SL), SL)])
    pltpu.sync_copy(cnt_sh.at[pl.ds(_al(sid * 2 * SL + SL), SL)], zb_v)
    pltpu.sync_copy(zb_v, counts_h.at[pl.ds(_al(cid * NGP + sid * 2 * SL + SL), SL)])


@functools.partial(
    pl.kernel,
    out_type=jax.ShapeDtypeStruct((NW * 16,), _i32),
    mesh=_mesh,
    compiler_params=pltpu.CompilerParams(needs_layout_passes=False),
    scratch_types=[
        pltpu.VMEM((SL,), _i32),
        pltpu.VMEM((SL,), _i32),
        pltpu.VMEM((16,), _i32),
    ],
)
def _k2(counts_h, sums_h, c0_v, c1_v, s_v):
    wid = _wid()
    pltpu.sync_copy(counts_h.at[pl.ds(_al(wid * SL), SL)], c0_v)
    pltpu.sync_copy(counts_h.at[pl.ds(_al(NGP + wid * SL), SL)], c1_v)

    def body(k, s):
        v = c0_v[pl.ds(k * 16, 16)] + c1_v[pl.ds(k * 16, 16)]
        flag = jnp.where(v > 0, 1, 0).astype(_i32)
        return s + jnp.sum(flag)
    total = lax.fori_loop(0, SL // 16, body, jnp.int32(0))
    s_v[pl.ds(0, 16)] = lax.broadcast(total, (16,))
    pltpu.sync_copy(s_v, sums_h.at[pl.ds(_al(wid * 16), 16)])


@functools.partial(
    pl.kernel,
    out_type=[
        jax.ShapeDtypeStruct((OUT_ROWS, D), jnp.float32),
        jax.ShapeDtypeStruct((NGP,), _i32),
    ],
    mesh=_mesh,
    compiler_params=pltpu.CompilerParams(needs_layout_passes=False),
    scratch_types=[
        pltpu.VMEM((SL,), _i32),
        pltpu.VMEM((SL,), _i32),
        pltpu.VMEM((SL,), _i32),
        pltpu.VMEM((PIR, 128), _i32),
        pltpu.VMEM((NW * 16,), _i32),
        pltpu.VMEM((128,), _i32),
        pltpu.VMEM((128,), _i32),
        pltpu.VMEM((128,), _i32),
        pltpu.VMEM((128,), _i32),
        pltpu.VMEM((128,), _i32),
        pltpu.VMEM((128, D), jnp.float32),
        pltpu.VMEM((128, D), jnp.float32),
        pltpu.VMEM((128, D), jnp.float32),
        pltpu.VMEM((128, D), jnp.float32),
        pltpu.VMEM((128, D), jnp.float32),
        pltpu.SemaphoreType.DMA,
        pltpu.SemaphoreType.DMA,
        pltpu.SemaphoreType.DMA,
        pltpu.SemaphoreType.DMA,
        pltpu.SemaphoreType.DMA,
        pltpu.SemaphoreType.DMA,
        pltpu.SemaphoreType.DMA,
        pltpu.SemaphoreType.DMA,
        pltpu.SemaphoreType.DMA,
    ],
)
def _k3(counts_h, sums_h, feat_h, out_h, ranks_h,
        c0_v, c1_v, ranks_v, comp_v, sums_v,
        oidx0, oidx1, oidx2, oidx3, bidx_v,
        rows0, rows1, rows2, rows3, pad_v,
        sg0, sg1, sg2, sg3, ss0, ss1, ss2, ss3, sem_p):
    wid = _wid()
    iota = lax.iota(_i32, 16)
    oidx = [oidx0, oidx1, oidx2, oidx3]
    rows = [rows0, rows1, rows2, rows3]
    sg = [sg0, sg1, sg2, sg3]
    ss = [ss0, ss1, ss2, ss3]

    pltpu.sync_copy(sums_h, sums_v)

    def pbody(i, carry):
        r, t = carry
        s_i = jnp.max(sums_v[pl.ds(i * 16, 16)])
        return (r + jnp.where(i < wid, s_i, 0), t + s_i)
    r0, num_unique = lax.fori_loop(0, NW, pbody, (jnp.int32(0), jnp.int32(0)))

    pltpu.sync_copy(feat_h.at[0], pad_v.at[0])
    frow = [pad_v[0, pl.ds(s * 16, 16)] for s in range(8)]

    def fbody(r, _):
        for s in range(8):
            pad_v[r, pl.ds(s * 16, 16)] = frow[s]
        return 0
    lax.fori_loop(1, 128, fbody, 0)

    nu_ceil = ((num_unique + 127) // 128) * 128
    bcnt = nu_ceil - num_unique

    @pl.when(wid == 0)
    def _():
        for s in range(8):
            off = s * 16 + iota
            bidx_v[pl.ds(s * 16, 16)] = jnp.where(
                off < bcnt, num_unique + off, TRASH)
        pltpu.async_copy(pad_v, out_h.at[bidx_v], sem_p)

    def pcond(st):
        j, n = st
        return nu_ceil + j * 128 < NTOT

    def pfill(st):
        j, n = st
        start = pl.multiple_of(nu_ceil + j * 128, 128)
        pltpu.async_copy(pad_v, out_h.at[pl.ds(start, 128)], sem_p)
        return (j + NW, n + 1)
    _, npad = lax.while_loop(pcond, pfill, (wid, jnp.int32(0)))

    def czero(k, _):
        comp_v[k // 8, pl.ds((k % 8) * 16, 16)] = jnp.zeros((16,), _i32)
        return 0
    lax.fori_loop(0, PI // 16, czero, 0)

    pltpu.sync_copy(counts_h.at[pl.ds(_al(wid * SL), SL)], c0_v)
    pltpu.sync_copy(counts_h.at[pl.ds(_al(NGP + wid * SL), SL)], c1_v)

    g0 = wid * SL
    a_off = lax.rem(r0, jnp.int32(128))
    row_base = r0 - a_off

    def rbody(k, acc):
        v = c0_v[pl.ds(k * 16, 16)] + c1_v[pl.ds(k * 16, 16)]
        flag_b = v > 0
        flag = jnp.where(flag_b, 1, 0).astype(_i32)
        incl = plsc.cumsum(flag)
        pos = acc + (incl - flag)
        g = g0 + k * 16 + iota
        q = a_off + pos
        plsc.store_scatter(comp_v, [q // 128, q % 128], g, mask=flag_b)
        ranks_v[pl.ds(k * 16, 16)] = r0 + pos
        return acc + jnp.sum(flag)
    cnt = lax.fori_loop(0, SL // 16, rbody, jnp.int32(0))
    pltpu.sync_copy(ranks_v, ranks_h.at[pl.ds(_al(wid * SL), SL)])

    qend = a_off + cnt
    nch = (qend + 127) // 128

    for b in range(2):
        @pl.when(b < nch)
        def _(b=b):
            pltpu.async_copy(feat_h.at[comp_v.at[b]], rows[b], sg[b])

    def grp(j, _):
        for b in range(4):
            k = j * 4 + b
            b2 = (b + 2) % 4

            @pl.when((k >= 2) & (k < nch + 2))
            def _(b2=b2):
                pltpu.make_async_copy(rows[b2],
                                      out_h.at[pl.ds(0, 128)], ss[b2]).wait()

            @pl.when(k + 2 < nch)
            def _(b2=b2, k=k):
                pltpu.async_copy(feat_h.at[comp_v.at[k + 2]], rows[b2], sg[b2])

            @pl.when(k < nch)
            def _(b=b, k=k):
                pltpu.make_async_copy(feat_h.at[pl.ds(0, 128)],
                                      rows[b], sg[b]).wait()
                full = (k * 128 >= a_off) & ((k + 1) * 128 <= qend)

                @pl.when(full)
                def _(b=b, k=k):
                    start = pl.multiple_of(row_base + k * 128, 8)
                    pltpu.async_copy(rows[b], out_h.at[pl.ds(start, 128)],
                                     ss[b])

                @pl.when(jnp.logical_not(full))
                def _(b=b, k=k):
                    for s in range(8):
                        off = k * 128 + s * 16 + iota
                        valid = (off >= a_off) & (off < qend)
                        oidx[b][pl.ds(s * 16, 16)] = jnp.where(
                            valid, row_base + off, TRASH)
                    pltpu.async_copy(rows[b], out_h.at[oidx[b]], ss[b])
        return 0
    lax.fori_loop(0, (nch + 5) // 4, grp, 0)

    def pdrain(j, _):
        pltpu.make_async_copy(pad_v, out_h.at[pl.ds(0, 128)], sem_p).wait()
        return 0
    lax.fori_loop(0, npad, pdrain, 0)

    @pl.when(wid == 0)
    def _():
        pltpu.make_async_copy(pad_v, out_h.at[pl.ds(0, 128)], sem_p).wait()


@functools.partial(
    pl.kernel,
    out_type=jax.ShapeDtypeStruct((NI,), _i32),
    mesh=_mesh,
    compiler_params=pltpu.CompilerParams(needs_layout_passes=False),
    scratch_types=[
        pltpu.VMEM((PI,), _i32),
        pltpu.VMEM((PI,), _i32),
        pltpu.SemaphoreType.DMA,
    ],
)
def _k4(gitems_h, ranks_h, inv_h, g_v, inv_v, sem):
    wid = _wid()
    pltpu.sync_copy(gitems_h.at[pl.ds(_al(wid * PI), PI)], g_v)

    pltpu.async_copy(ranks_h.at[g_v], inv_v, sem).wait()
    pltpu.sync_copy(inv_v, inv_h.at[pl.ds(_al(wid * PI), PI)])


@jax.jit
def kernel(client_ids, item_ids, item_id2graph_id, client_id2graph_id, node_feat):
    item_flat = item_ids.reshape(NI).astype(_i32)
    gitems, counts = _k1(item_flat, client_ids.astype(_i32),
                         item_id2graph_id.astype(_i32),
                         client_id2graph_id.astype(_i32))
    sums = _k2(counts)
    feats, ranks = _k3(counts, sums, node_feat)
    inv = _k4(gitems, ranks)
    return feats[:NTOT], inv.reshape(item_ids.shape)


# speedup vs baseline: 1.3341x; 1.3341x over previous
import functools

import jax
import jax.numpy as jnp
from jax import lax
from jax.experimental import pallas as pl
from jax.experimental.pallas import tpu as pltpu
from jax.experimental.pallas import tpu_sc as plsc

NC = 2
NS = 16
NW = NC * NS

NG = 200000
SL = 6256
NGP = NW * SL

NI = 204800
PI = 6400
PIR = 50
NCL = 1024
PC = 32

NTOT = NI + NCL
D = 128

OUT_ROWS = NTOT + 256
TRASH = NTOT + 128

_mesh = plsc.VectorSubcoreMesh(
    core_axis_name="c", subcore_axis_name="s", num_cores=NC, num_subcores=NS
)

_i32 = jnp.int32


def _wid():
    return lax.axis_index("s") * NC + lax.axis_index("c")


def _al(x):
    return pl.multiple_of(x, 8)


@functools.partial(
    pl.kernel,
    out_type=[
        jax.ShapeDtypeStruct((NI,), _i32),
        jax.ShapeDtypeStruct((NC * NGP,), _i32),
    ],
    mesh=_mesh,
    compiler_params=pltpu.CompilerParams(needs_layout_passes=False, skip_device_barrier=True),
    scratch_types=[
        pltpu.VMEM((PI,), _i32),
        pltpu.VMEM((PI,), _i32),
        pltpu.VMEM((PC,), _i32),
        pltpu.VMEM((PC,), _i32),
        pltpu.VMEM((PI,), _i32),
        pltpu.VMEM((SL,), _i32),
        pltpu.VMEM_SHARED((NGP,), _i32),
        pltpu.SemaphoreType.DMA,
        pltpu.SemaphoreType.DMA,
    ],
)
def _k1(item_ids_h, client_ids_h, itab_h, ctab_h, gitems_h, counts_h,
        iidx_v, gitem_v, cidx_v, gcli_v, ones_v, zb_v, cnt_sh, sem, sem2):
    cid = lax.axis_index("c")
    sid = lax.axis_index("s")
    wid = sid * NC + cid

    def zfill(k, _):
        zb_v[pl.ds(k * 16, 16)] = jnp.zeros((16,), _i32)
        return 0
    lax.fori_loop(0, SL // 16, zfill, 0)
    pltpu.sync_copy(zb_v, cnt_sh.at[pl.ds(_al(sid * 2 * SL), SL)])
    pltpu.sync_copy(zb_v, cnt_sh.at[pl.ds(_al(sid * 2 * SL + SL), SL)])

    def ofill(k, _):
        ones_v[pl.ds(k * 16, 16)] = jnp.ones((16,), _i32)
        return 0
    lax.fori_loop(0, PI // 16, ofill, 0)

    pltpu.sync_copy(item_ids_h.at[pl.ds(_al(wid * PI), PI)], iidx_v)
    gdesc = pltpu.async_copy(itab_h.at[iidx_v], gitem_v, sem)

    pltpu.sync_copy(client_ids_h.at[pl.ds(_al(wid * PC), PC)], cidx_v)
    pltpu.async_copy(ctab_h.at[cidx_v], gcli_v, sem2).wait()

    gdesc.wait()
    pltpu.async_copy(gitem_v, gitems_h.at[pl.ds(_al(wid * PI), PI)], sem2)

    plsc.subcore_barrier()

    pltpu.async_copy(ones_v, cnt_sh.at[gitem_v], sem, add=True)
    pltpu.async_copy(ones_v.at[pl.ds(0, PC)], cnt_sh.at[gcli_v], sem,
                     add=True)
    pltpu.make_async_copy(itab_h.at[pl.ds(0, PC)], cidx_v, sem).wait()
    pltpu.make_async_copy(itab_h.at[pl.ds(0, PI)], ones_v, sem).wait()
    pltpu.make_async_copy(gitem_v, gitems_h.at[pl.ds(0, PI)], sem2).wait()

    plsc.subcore_barrier()

    pltpu.sync_copy(cnt_sh.at[pl.ds(_al(sid * 2 * SL), SL)], zb_v)
    pltpu.sync_copy(zb_v, counts_h.at[pl.ds(_al(cid * NGP + sid * 2 * SL), SL)])
    pltpu.sync_copy(cnt_sh.at[pl.ds(_al(sid * 2 * SL + SL), SL)], zb_v)
    pltpu.sync_copy(zb_v, counts_h.at[pl.ds(_al(cid * NGP + sid * 2 * SL + SL), SL)])


@functools.partial(
    pl.kernel,
    out_type=jax.ShapeDtypeStruct((NW * 16,), _i32),
    mesh=_mesh,
    compiler_params=pltpu.CompilerParams(needs_layout_passes=False, skip_device_barrier=True),
    scratch_types=[
        pltpu.VMEM((SL,), _i32),
        pltpu.VMEM((SL,), _i32),
        pltpu.VMEM((16,), _i32),
    ],
)
def _k2(counts_h, sums_h, c0_v, c1_v, s_v):
    wid = _wid()
    pltpu.sync_copy(counts_h.at[pl.ds(_al(wid * SL), SL)], c0_v)
    pltpu.sync_copy(counts_h.at[pl.ds(_al(NGP + wid * SL), SL)], c1_v)

    def body(k, s):
        v = c0_v[pl.ds(k * 16, 16)] + c1_v[pl.ds(k * 16, 16)]
        flag = jnp.where(v > 0, 1, 0).astype(_i32)
        return s + jnp.sum(flag)
    total = lax.fori_loop(0, SL // 16, body, jnp.int32(0))
    s_v[pl.ds(0, 16)] = lax.broadcast(total, (16,))
    pltpu.sync_copy(s_v, sums_h.at[pl.ds(_al(wid * 16), 16)])


@functools.partial(
    pl.kernel,
    out_type=[
        jax.ShapeDtypeStruct((OUT_ROWS, D), jnp.float32),
        jax.ShapeDtypeStruct((NGP,), _i32),
    ],
    mesh=_mesh,
    compiler_params=pltpu.CompilerParams(needs_layout_passes=False, skip_device_barrier=True),
    scratch_types=[
        pltpu.VMEM((SL,), _i32),
        pltpu.VMEM((SL,), _i32),
        pltpu.VMEM((SL,), _i32),
        pltpu.VMEM((PIR, 128), _i32),
        pltpu.VMEM((NW * 16,), _i32),
        pltpu.VMEM((128,), _i32),
        pltpu.VMEM((128,), _i32),
        pltpu.VMEM((128,), _i32),
        pltpu.VMEM((128,), _i32),
        pltpu.VMEM((128,), _i32),
        pltpu.VMEM((128, D), jnp.float32),
        pltpu.VMEM((128, D), jnp.float32),
        pltpu.VMEM((128, D), jnp.float32),
        pltpu.VMEM((128, D), jnp.float32),
        pltpu.VMEM((128, D), jnp.float32),
        pltpu.SemaphoreType.DMA,
        pltpu.SemaphoreType.DMA,
        pltpu.SemaphoreType.DMA,
        pltpu.SemaphoreType.DMA,
        pltpu.SemaphoreType.DMA,
        pltpu.SemaphoreType.DMA,
        pltpu.SemaphoreType.DMA,
        pltpu.SemaphoreType.DMA,
        pltpu.SemaphoreType.DMA,
    ],
)
def _k3(counts_h, sums_h, feat_h, out_h, ranks_h,
        c0_v, c1_v, ranks_v, comp_v, sums_v,
        oidx0, oidx1, oidx2, oidx3, bidx_v,
        rows0, rows1, rows2, rows3, pad_v,
        sg0, sg1, sg2, sg3, ss0, ss1, ss2, ss3, sem_p):
    wid = _wid()
    iota = lax.iota(_i32, 16)
    oidx = [oidx0, oidx1, oidx2, oidx3]
    rows = [rows0, rows1, rows2, rows3]
    sg = [sg0, sg1, sg2, sg3]
    ss = [ss0, ss1, ss2, ss3]

    pltpu.sync_copy(sums_h, sums_v)

    def pbody(i, carry):
        r, t = carry
        s_i = jnp.max(sums_v[pl.ds(i * 16, 16)])
        return (r + jnp.where(i < wid, s_i, 0), t + s_i)
    r0, num_unique = lax.fori_loop(0, NW, pbody, (jnp.int32(0), jnp.int32(0)))

    pltpu.sync_copy(feat_h.at[0], pad_v.at[0])
    frow = [pad_v[0, pl.ds(s * 16, 16)] for s in range(8)]

    def fbody(r, _):
        for s in range(8):
            pad_v[r, pl.ds(s * 16, 16)] = frow[s]
        return 0
    lax.fori_loop(1, 128, fbody, 0)

    nu_ceil = ((num_unique + 127) // 128) * 128
    bcnt = nu_ceil - num_unique

    @pl.when(wid == 0)
    def _():
        for s in range(8):
            off = s * 16 + iota
            bidx_v[pl.ds(s * 16, 16)] = jnp.where(
                off < bcnt, num_unique + off, TRASH)
        pltpu.async_copy(pad_v, out_h.at[bidx_v], sem_p)

    def pcond(st):
        j, n = st
        return nu_ceil + j * 128 < NTOT

    def pfill(st):
        j, n = st
        start = pl.multiple_of(nu_ceil + j * 128, 128)
        pltpu.async_copy(pad_v, out_h.at[pl.ds(start, 128)], sem_p)
        return (j + NW, n + 1)
    _, npad = lax.while_loop(pcond, pfill, (wid, jnp.int32(0)))

    def czero(k, _):
        comp_v[k // 8, pl.ds((k % 8) * 16, 16)] = jnp.zeros((16,), _i32)
        return 0
    lax.fori_loop(0, PI // 16, czero, 0)

    pltpu.sync_copy(counts_h.at[pl.ds(_al(wid * SL), SL)], c0_v)
    pltpu.sync_copy(counts_h.at[pl.ds(_al(NGP + wid * SL), SL)], c1_v)

    g0 = wid * SL

    def rbody(k, acc):
        v = c0_v[pl.ds(k * 16, 16)] + c1_v[pl.ds(k * 16, 16)]
        flag_b = v > 0
        flag = jnp.where(flag_b, 1, 0).astype(_i32)
        incl = plsc.cumsum(flag)
        pos = acc + (incl - flag)
        g = g0 + k * 16 + iota
        plsc.store_scatter(comp_v, [pos // 128, pos % 128], g, mask=flag_b)
        ranks_v[pl.ds(k * 16, 16)] = r0 + pos
        return acc + jnp.sum(flag)
    cnt = lax.fori_loop(0, SL // 16, rbody, jnp.int32(0))
    pltpu.sync_copy(ranks_v, ranks_h.at[pl.ds(_al(wid * SL), SL)])

    nch = (cnt + 127) // 128

    for b in range(2):
        @pl.when(b < nch)
        def _(b=b):
            pltpu.async_copy(feat_h.at[comp_v.at[b]], rows[b], sg[b])

    def grp(j, _):
        for b in range(4):
            k = j * 4 + b
            b2 = (b + 2) % 4

            @pl.when((k >= 2) & (k < nch + 2))
            def _(b2=b2):
                pltpu.make_async_copy(rows[b2],
                                      out_h.at[pl.ds(0, 128)], ss[b2]).wait()

            @pl.when(k + 2 < nch)
            def _(b2=b2, k=k):
                pltpu.async_copy(feat_h.at[comp_v.at[k + 2]], rows[b2], sg[b2])

            @pl.when(k < nch)
            def _(b=b, k=k):
                pltpu.make_async_copy(feat_h.at[pl.ds(0, 128)],
                                      rows[b], sg[b]).wait()
                for s in range(8):
                    off = k * 128 + s * 16 + iota
                    oidx[b][pl.ds(s * 16, 16)] = jnp.where(
                        off < cnt, r0 + off, TRASH)
                pltpu.async_copy(rows[b], out_h.at[oidx[b]], ss[b])
        return 0
    lax.fori_loop(0, (nch + 5) // 4, grp, 0)

    def pdrain(j, _):
        pltpu.make_async_copy(pad_v, out_h.at[pl.ds(0, 128)], sem_p).wait()
        return 0
    lax.fori_loop(0, npad, pdrain, 0)

    @pl.when(wid == 0)
    def _():
        pltpu.make_async_copy(pad_v, out_h.at[pl.ds(0, 128)], sem_p).wait()


@functools.partial(
    pl.kernel,
    out_type=jax.ShapeDtypeStruct((NI,), _i32),
    mesh=_mesh,
    compiler_params=pltpu.CompilerParams(needs_layout_passes=False, skip_device_barrier=True),
    scratch_types=[
        pltpu.VMEM((PI,), _i32),
        pltpu.VMEM((PI,), _i32),
        pltpu.SemaphoreType.DMA,
    ],
)
def _k4(gitems_h, ranks_h, inv_h, g_v, inv_v, sem):
    wid = _wid()
    pltpu.sync_copy(gitems_h.at[pl.ds(_al(wid * PI), PI)], g_v)

    pltpu.async_copy(ranks_h.at[g_v], inv_v, sem).wait()
    pltpu.sync_copy(inv_v, inv_h.at[pl.ds(_al(wid * PI), PI)])


@jax.jit
def kernel(client_ids, item_ids, item_id2graph_id, client_id2graph_id, node_feat):
    item_flat = item_ids.reshape(NI).astype(_i32)
    gitems, counts = _k1(item_flat, client_ids.astype(_i32),
                         item_id2graph_id.astype(_i32),
                         client_id2graph_id.astype(_i32))
    sums = _k2(counts)
    feats, ranks = _k3(counts, sums, node_feat)
    inv = _k4(gitems, ranks)
    return feats[:NTOT], inv.reshape(item_ids.shape)


# speedup vs baseline: 1.9346x; 1.4501x over previous
import functools

import jax
import jax.numpy as jnp
from jax import lax
from jax.experimental import pallas as pl
from jax.experimental.pallas import tpu as pltpu
from jax.experimental.pallas import tpu_sc as plsc

NC = 2
NS = 16
NW = NC * NS

NG = 200000
SL = 6256
NGP = NW * SL

NI = 204800
PI = 6400
PIR = 50
NCL = 1024
PC = 32

NTOT = NI + NCL
D = 128

OUT_ROWS = NTOT
PADB = NTOT - 128

_mesh = plsc.VectorSubcoreMesh(
    core_axis_name="c", subcore_axis_name="s", num_cores=NC, num_subcores=NS
)

_i32 = jnp.int32


def _wid():
    return lax.axis_index("s") * NC + lax.axis_index("c")


def _al(x):
    return pl.multiple_of(x, 8)


@functools.partial(
    pl.kernel,
    out_type=[
        jax.ShapeDtypeStruct((NI,), _i32),
        jax.ShapeDtypeStruct((NC * NGP,), _i32),
    ],
    mesh=_mesh,
    compiler_params=pltpu.CompilerParams(needs_layout_passes=False),
    scratch_types=[
        pltpu.VMEM((PI,), _i32),
        pltpu.VMEM((PI,), _i32),
        pltpu.VMEM((PC,), _i32),
        pltpu.VMEM((PC,), _i32),
        pltpu.VMEM((PI,), _i32),
        pltpu.VMEM((SL,), _i32),
        pltpu.VMEM_SHARED((NGP,), _i32),
        pltpu.SemaphoreType.DMA,
        pltpu.SemaphoreType.DMA,
    ],
)
def _k1(item_ids_h, client_ids_h, itab_h, ctab_h, gitems_h, counts_h,
        iidx_v, gitem_v, cidx_v, gcli_v, ones_v, zb_v, cnt_sh, sem, sem2):
    cid = lax.axis_index("c")
    sid = lax.axis_index("s")
    wid = sid * NC + cid

    def zfill(k, _):
        zb_v[pl.ds(k * 16, 16)] = jnp.zeros((16,), _i32)
        return 0
    lax.fori_loop(0, SL // 16, zfill, 0)
    pltpu.sync_copy(zb_v, cnt_sh.at[pl.ds(_al(sid * 2 * SL), SL)])
    pltpu.sync_copy(zb_v, cnt_sh.at[pl.ds(_al(sid * 2 * SL + SL), SL)])

    def ofill(k, _):
        ones_v[pl.ds(k * 16, 16)] = jnp.ones((16,), _i32)
        return 0
    lax.fori_loop(0, PI // 16, ofill, 0)

    pltpu.sync_copy(item_ids_h.at[pl.ds(_al(wid * PI), PI)], iidx_v)
    gdesc = pltpu.async_copy(itab_h.at[iidx_v], gitem_v, sem)

    pltpu.sync_copy(client_ids_h.at[pl.ds(_al(wid * PC), PC)], cidx_v)
    pltpu.async_copy(ctab_h.at[cidx_v], gcli_v, sem2).wait()

    gdesc.wait()
    pltpu.async_copy(gitem_v, gitems_h.at[pl.ds(_al(wid * PI), PI)], sem2)

    plsc.subcore_barrier()

    pltpu.async_copy(ones_v, cnt_sh.at[gitem_v], sem, add=True)
    pltpu.async_copy(ones_v.at[pl.ds(0, PC)], cnt_sh.at[gcli_v], sem,
                     add=True)
    pltpu.make_async_copy(itab_h.at[pl.ds(0, PC)], cidx_v, sem).wait()
    pltpu.make_async_copy(itab_h.at[pl.ds(0, PI)], ones_v, sem).wait()
    pltpu.make_async_copy(gitem_v, gitems_h.at[pl.ds(0, PI)], sem2).wait()

    plsc.subcore_barrier()

    pltpu.sync_copy(cnt_sh.at[pl.ds(_al(sid * 2 * SL), SL)], zb_v)
    pltpu.sync_copy(zb_v, counts_h.at[pl.ds(_al(cid * NGP + sid * 2 * SL), SL)])
    pltpu.sync_copy(cnt_sh.at[pl.ds(_al(sid * 2 * SL + SL), SL)], zb_v)
    pltpu.sync_copy(zb_v, counts_h.at[pl.ds(_al(cid * NGP + sid * 2 * SL + SL), SL)])


@functools.partial(
    pl.kernel,
    out_type=jax.ShapeDtypeStruct((NW * 16,), _i32),
    mesh=_mesh,
    compiler_params=pltpu.CompilerParams(needs_layout_passes=False),
    scratch_types=[
        pltpu.VMEM((SL,), _i32),
        pltpu.VMEM((SL,), _i32),
        pltpu.VMEM((16,), _i32),
    ],
)
def _k2(counts_h, sums_h, c0_v, c1_v, s_v):
    wid = _wid()
    pltpu.sync_copy(counts_h.at[pl.ds(_al(wid * SL), SL)], c0_v)
    pltpu.sync_copy(counts_h.at[pl.ds(_al(NGP + wid * SL), SL)], c1_v)

    def body(k, s):
        v = c0_v[pl.ds(k * 16, 16)] + c1_v[pl.ds(k * 16, 16)]
        flag = jnp.where(v > 0, 1, 0).astype(_i32)
        return s + jnp.sum(flag)
    total = lax.fori_loop(0, SL // 16, body, jnp.int32(0))
    s_v[pl.ds(0, 16)] = lax.broadcast(total, (16,))
    pltpu.sync_copy(s_v, sums_h.at[pl.ds(_al(wid * 16), 16)])


@functools.partial(
    pl.kernel,
    out_type=[
        jax.ShapeDtypeStruct((OUT_ROWS, D), jnp.float32),
        jax.ShapeDtypeStruct((NGP,), _i32),
    ],
    mesh=_mesh,
    compiler_params=pltpu.CompilerParams(needs_layout_passes=False),
    scratch_types=[
        pltpu.VMEM((SL,), _i32),
        pltpu.VMEM((SL,), _i32),
        pltpu.VMEM((SL,), _i32),
        pltpu.VMEM((PIR, 128), _i32),
        pltpu.VMEM((NW * 16,), _i32),
        pltpu.VMEM((128,), _i32),
        pltpu.VMEM((128,), _i32),
        pltpu.VMEM((128,), _i32),
        pltpu.VMEM((128,), _i32),
        pltpu.VMEM((128,), _i32),
        pltpu.VMEM((128, D), jnp.float32),
        pltpu.VMEM((128, D), jnp.float32),
        pltpu.VMEM((128, D), jnp.float32),
        pltpu.VMEM((128, D), jnp.float32),
        pltpu.VMEM((128, D), jnp.float32),
        pltpu.SemaphoreType.DMA,
        pltpu.SemaphoreType.DMA,
        pltpu.SemaphoreType.DMA,
        pltpu.SemaphoreType.DMA,
        pltpu.SemaphoreType.DMA,
        pltpu.SemaphoreType.DMA,
        pltpu.SemaphoreType.DMA,
        pltpu.SemaphoreType.DMA,
        pltpu.SemaphoreType.DMA,
    ],
)
def _k3(counts_h, sums_h, feat_h, out_h, ranks_h,
        c0_v, c1_v, ranks_v, comp_v, sums_v,
        oidx0, oidx1, oidx2, oidx3, bidx_v,
        rows0, rows1, rows2, rows3, pad_v,
        sg0, sg1, sg2, sg3, ss0, ss1, ss2, ss3, sem_p):
    wid = _wid()
    iota = lax.iota(_i32, 16)
    oidx = [oidx0, oidx1, oidx2, oidx3]
    rows = [rows0, rows1, rows2, rows3]
    sg = [sg0, sg1, sg2, sg3]
    ss = [ss0, ss1, ss2, ss3]

    pltpu.sync_copy(sums_h, sums_v)

    def pbody(i, carry):
        r, t = carry
        s_i = jnp.max(sums_v[pl.ds(i * 16, 16)])
        return (r + jnp.where(i < wid, s_i, 0), t + s_i)
    r0, num_unique = lax.fori_loop(0, NW, pbody, (jnp.int32(0), jnp.int32(0)))

    pltpu.sync_copy(feat_h.at[0], pad_v.at[0])
    frow = [pad_v[0, pl.ds(s * 16, 16)] for s in range(8)]

    def fbody(r, _):
        for s in range(8):
            pad_v[r, pl.ds(s * 16, 16)] = frow[s]
        return 0
    lax.fori_loop(1, 128, fbody, 0)

    nu_ceil = ((num_unique + 127) // 128) * 128
    bcnt = nu_ceil - num_unique

    @pl.when(wid == 0)
    def _():
        for s in range(8):
            off = s * 16 + iota
            bidx_v[pl.ds(s * 16, 16)] = jnp.where(
                off < bcnt, num_unique + off, PADB + off)
        pltpu.async_copy(pad_v, out_h.at[bidx_v], sem_p)

    def pcond(st):
        j, n = st
        return nu_ceil + j * 128 < NTOT

    def pfill(st):
        j, n = st
        start = pl.multiple_of(jnp.minimum(nu_ceil + j * 128, PADB), 128)
        pltpu.async_copy(pad_v, out_h.at[pl.ds(start, 128)], sem_p)
        return (j + NW, n + 1)
    _, npad = lax.while_loop(pcond, pfill, (wid, jnp.int32(0)))

    def czero(k, _):
        comp_v[k // 8, pl.ds((k % 8) * 16, 16)] = jnp.zeros((16,), _i32)
        return 0
    lax.fori_loop(0, PI // 16, czero, 0)

    pltpu.sync_copy(counts_h.at[pl.ds(_al(wid * SL), SL)], c0_v)
    pltpu.sync_copy(counts_h.at[pl.ds(_al(NGP + wid * SL), SL)], c1_v)

    g0 = wid * SL

    def rbody(k, acc):
        v = c0_v[pl.ds(k * 16, 16)] + c1_v[pl.ds(k * 16, 16)]
        flag_b = v > 0
        flag = jnp.where(flag_b, 1, 0).astype(_i32)
        incl = plsc.cumsum(flag)
        pos = acc + (incl - flag)
        g = g0 + k * 16 + iota
        plsc.store_scatter(comp_v, [pos // 128, pos % 128], g, mask=flag_b)
        ranks_v[pl.ds(k * 16, 16)] = r0 + pos
        return acc + jnp.sum(flag)
    cnt = lax.fori_loop(0, SL // 16, rbody, jnp.int32(0))
    pltpu.sync_copy(ranks_v, ranks_h.at[pl.ds(_al(wid * SL), SL)])

    nch = (cnt + 127) // 128

    for b in range(2):
        @pl.when(b < nch)
        def _(b=b):
            pltpu.async_copy(feat_h.at[comp_v.at[b]], rows[b], sg[b])

    def grp(j, _):
        for b in range(4):
            k = j * 4 + b
            b2 = (b + 2) % 4

            @pl.when((k >= 2) & (k < nch + 2))
            def _(b2=b2):
                pltpu.make_async_copy(rows[b2],
                                      out_h.at[pl.ds(0, 128)], ss[b2]).wait()

            @pl.when(k + 2 < nch)
            def _(b2=b2, k=k):
                pltpu.async_copy(feat_h.at[comp_v.at[k + 2]], rows[b2], sg[b2])

            @pl.when(k < nch)
            def _(b=b, k=k):
                pltpu.make_async_copy(feat_h.at[pl.ds(0, 128)],
                                      rows[b], sg[b]).wait()
                for s in range(8):
                    off = k * 128 + s * 16 + iota
                    oidx[b][pl.ds(s * 16, 16)] = jnp.where(
                        off < cnt, r0 + off, PADB + s * 16 + iota)
                pltpu.async_copy(rows[b], out_h.at[oidx[b]], ss[b])
        return 0
    lax.fori_loop(0, (nch + 5) // 4, grp, 0)

    def pdrain(j, _):
        pltpu.make_async_copy(pad_v, out_h.at[pl.ds(0, 128)], sem_p).wait()
        return 0
    lax.fori_loop(0, npad, pdrain, 0)

    @pl.when(wid == 0)
    def _():
        pltpu.make_async_copy(pad_v, out_h.at[pl.ds(0, 128)], sem_p).wait()


@functools.partial(
    pl.kernel,
    out_type=jax.ShapeDtypeStruct((NI,), _i32),
    mesh=_mesh,
    compiler_params=pltpu.CompilerParams(needs_layout_passes=False),
    scratch_types=[
        pltpu.VMEM((PI,), _i32),
        pltpu.VMEM((PI,), _i32),
        pltpu.SemaphoreType.DMA,
    ],
)
def _k4(gitems_h, ranks_h, inv_h, g_v, inv_v, sem):
    wid = _wid()
    pltpu.sync_copy(gitems_h.at[pl.ds(_al(wid * PI), PI)], g_v)

    pltpu.async_copy(ranks_h.at[g_v], inv_v, sem).wait()
    pltpu.sync_copy(inv_v, inv_h.at[pl.ds(_al(wid * PI), PI)])


@jax.jit
def kernel(client_ids, item_ids, item_id2graph_id, client_id2graph_id, node_feat):
    item_flat = item_ids.reshape(NI).astype(_i32)
    gitems, counts = _k1(item_flat, client_ids.astype(_i32),
                         item_id2graph_id.astype(_i32),
                         client_id2graph_id.astype(_i32))
    sums = _k2(counts)
    feats, ranks = _k3(counts, sums, node_feat)
    inv = _k4(gitems, ranks)
    return feats, inv.reshape(item_ids.shape)
